# baseline (speedup 1.0000x reference)
import functools

import jax
import jax.numpy as jnp
from jax import lax
from jax.experimental import pallas as pl
from jax.experimental.pallas import tpu as pltpu
from jax.experimental.pallas import tpu_sc as plsc

NC = 2
NS = 16
NW = NC * NS
CH = 128
RB = 6


def _mesh():
    return plsc.VectorSubcoreMesh(core_axis_name="c", subcore_axis_name="s")


def _make_deg(n, kmc):
    chunk = 1000
    assert chunk % 8 == 0 and n % chunk == 0
    ntz = n // chunk
    zlen = -(-chunk // 16) * 16

    def body(dst3, ew3, out, dst2, ew2, zbuf, acc):
        c = lax.axis_index("c")
        s = lax.axis_index("s")
        w = c * NS + s
        pltpu.sync_copy(dst3.at[w], dst2)
        pltpu.sync_copy(ew3.at[w], ew2)

        def zb(i, _):
            zbuf[pl.ds(i * 16, 16)] = jnp.zeros((16,), jnp.float32)
            return 0
        lax.fori_loop(0, zlen // 16, zb, 0, unroll=8)

        @pl.when(s < ntz)
        def _():
            pltpu.sync_copy(zbuf.at[pl.ds(0, chunk)],
                            acc.at[pl.ds(s * chunk, chunk)])
        plsc.subcore_barrier()

        def scat(j, _):
            pltpu.sync_copy(ew2.at[j], acc.at[dst2.at[j]], add=True)
            return 0
        lax.fori_loop(0, kmc, scat, 0)
        plsc.subcore_barrier()

        @pl.when(s < ntz)
        def _():
            pltpu.sync_copy(acc.at[pl.ds(s * chunk, chunk)],
                            zbuf.at[pl.ds(0, chunk)])
            pltpu.sync_copy(zbuf.at[pl.ds(0, chunk)],
                            out.at[pl.ds(c * n + s * chunk, chunk)])

    return pl.kernel(
        body,
        out_type=jax.ShapeDtypeStruct((NC * n,), jnp.float32),
        mesh=_mesh(),
        scratch_types=[
            pltpu.VMEM((kmc, 128), jnp.int32),
            pltpu.VMEM((kmc, 128), jnp.float32),
            pltpu.VMEM((zlen,), jnp.float32),
            pltpu.VMEM_SHARED((n,), jnp.float32),
        ],
    )


def _make_agg(n, kmc, wd, nh):
    rows = (n // NS) // 8 * 8
    rem = n - NS * rows
    assert kmc % RB == 0 and rem % 8 == 0

    def body(src3, dst3, ew3, h, out, src2, dst2, ew2, b0, b1, b2, b3, b4,
             b5, acc, gsem, ssem):
        c = lax.axis_index("c")
        s = lax.axis_index("s")
        w = c * NS + s
        bufs = (b0, b1, b2, b3, b4, b5)
        pltpu.sync_copy(src3.at[w], src2)
        pltpu.sync_copy(dst3.at[w], dst2)
        pltpu.sync_copy(ew3.at[w], ew2)

        def zb(e, _):
            for g in range(wd // 16):
                b0[e, pl.ds(g * 16, 16)] = jnp.zeros((16,), jnp.float32)
            return 0
        lax.fori_loop(0, CH, zb, 0)

        def zero_acc():
            def zrange(base, nr):
                off = 0
                while off < nr:
                    sz = min(CH, nr - off)
                    pltpu.sync_copy(b0.at[pl.ds(0, sz)],
                                    acc.at[pl.ds(base + off, sz)])
                    off += sz
            zrange(s * rows, rows)

            @pl.when(s == 0)
            def _():
                zrange(NS * rows, rem)

        def scale(j, buf):
            ewrow = ew2.at[j]

            def tscale(t, _):
                ewv = ewrow[pl.ds(t * 16, 16)]
                bt = buf.at[pl.ds(t * 16, 16)]
                for i in range(16):
                    bv = ewv.at[jnp.full((16,), i, jnp.int32)].get(
                        mode="promise_in_bounds")
                    for g in range(wd // 16):
                        sl = (i, pl.ds(g * 16, 16))
                        bt[sl] = bt[sl] * bv
                return 0
            lax.fori_loop(0, CH // 16, tscale, 0, unroll=2)

        for f in range(nh):
            hf = h.at[f]
            zero_acc()
            plsc.subcore_barrier()

            def gfire(j, buf):
                pltpu.async_copy(hf.at[src2.at[j]], buf, gsem)

            def gwait(j, buf):
                pltpu.make_async_copy(hf.at[src2.at[j]], buf, gsem).wait()

            def sfire(j, buf):
                pltpu.async_copy(buf, acc.at[dst2.at[j]], ssem, add=True)

            def swait(j, buf):
                pltpu.make_async_copy(buf, acc.at[dst2.at[j]], ssem).wait()

            gfire(0, b0)
            gfire(1, b1)

            def step(i, _):
                for k in range(RB):
                    j = RB * i + k
                    buf = bufs[k]
                    nbuf = bufs[(k + 2) % RB]
                    gwait(j, buf)
                    scale(j, buf)
                    sfire(j, buf)

                    @pl.when(jnp.logical_and(j >= RB - 2, j + 2 < kmc))
                    def _():
                        swait(j - (RB - 2), nbuf)

                    @pl.when(j + 2 < kmc)
                    def _():
                        gfire(j + 2, nbuf)
                return 0
            lax.fori_loop(0, kmc // RB, step, 0)
            for j in range(kmc - RB, kmc):
                swait(j, bufs[j % RB])
            plsc.subcore_barrier()

            def dump(base, nr):
                off = 0
                while off < nr:
                    sz = min(CH, nr - off)
                    pltpu.sync_copy(acc.at[pl.ds(base + off, sz)],
                                    b0.at[pl.ds(0, sz)])
                    pltpu.sync_copy(b0.at[pl.ds(0, sz)],
                                    out.at[f, c, pl.ds(base + off, sz)])
                    off += sz
            dump(s * rows, rows)

            @pl.when(s == 0)
            def _():
                dump(NS * rows, rem)
            if f + 1 < nh:
                lax.fori_loop(0, CH, zb, 0)
                plsc.subcore_barrier()

    return pl.kernel(
        body,
        out_type=jax.ShapeDtypeStruct((nh, NC, n, wd), jnp.float32),
        mesh=_mesh(),
        compiler_params=pltpu.CompilerParams(use_tc_tiling_on_sc=False),
        scratch_types=[
            pltpu.VMEM((kmc, CH), jnp.int32),
            pltpu.VMEM((kmc, CH), jnp.int32),
            pltpu.VMEM((kmc, CH), jnp.float32),
            pltpu.VMEM((CH, wd), jnp.float32),
            pltpu.VMEM((CH, wd), jnp.float32),
            pltpu.VMEM((CH, wd), jnp.float32),
            pltpu.VMEM((CH, wd), jnp.float32),
            pltpu.VMEM((CH, wd), jnp.float32),
            pltpu.VMEM((CH, wd), jnp.float32),
            pltpu.VMEM_SHARED((n, wd), jnp.float32),
            pltpu.SemaphoreType.DMA,
            pltpu.SemaphoreType.DMA,
        ],
    )


_PREC = lax.Precision.HIGHEST


def _tc1_body(degp, emb, w1, h1s, dinv_out):
    deg = degp[0] + degp[1]
    dinv = jnp.where(deg > 0, lax.rsqrt(deg), 0.0)
    hm = jnp.dot(emb[...], w1[...], preferred_element_type=jnp.float32,
                 precision=_PREC)
    hm = hm * dinv
    half = hm.shape[1] // 2
    h1s[0] = hm[:, :half]
    h1s[1] = hm[:, half:]
    dinv_out[...] = dinv


def _tc2_body(p, dinv_in, b1, w2, h2):
    dinv = dinv_in[...]
    p01 = jnp.concatenate([p[0, 0] + p[0, 1], p[1, 0] + p[1, 1]], axis=1)
    x = p01 * dinv + b1[...]
    x = jnp.maximum(x, 0.0)
    hm = jnp.dot(x, w2[...], preferred_element_type=jnp.float32,
                 precision=_PREC)
    h2[...] = hm * dinv


def _tc3_body(q, dinv_in, b2, out):
    out[...] = (q[0] + q[1]) * dinv_in[...] + b2[...]


def _tc1(degp, emb, w1, blk):
    n, hdim = emb.shape
    grid = n // blk
    return pl.pallas_call(
        _tc1_body,
        grid=(grid,),
        in_specs=[
            pl.BlockSpec((NC, blk, 1), lambda i: (0, i, 0)),
            pl.BlockSpec((blk, hdim), lambda i: (i, 0)),
            pl.BlockSpec((hdim, hdim), lambda i: (0, 0)),
        ],
        out_specs=[
            pl.BlockSpec((2, blk, hdim // 2), lambda i: (0, i, 0)),
            pl.BlockSpec((blk, 1), lambda i: (i, 0)),
        ],
        out_shape=[
            jax.ShapeDtypeStruct((2, n, hdim // 2), jnp.float32),
            jax.ShapeDtypeStruct((n, 1), jnp.float32),
        ],
    )(degp[..., None], emb, w1)


def _tc2(p, dinv, b1, w2p, blk):
    _, _, n, hh = p.shape
    hdim = 2 * hh
    wd2 = w2p.shape[1]
    grid = n // blk
    return pl.pallas_call(
        _tc2_body,
        grid=(grid,),
        in_specs=[
            pl.BlockSpec((2, NC, blk, hh), lambda i: (0, 0, i, 0)),
            pl.BlockSpec((blk, 1), lambda i: (i, 0)),
            pl.BlockSpec((1, hdim), lambda i: (0, 0)),
            pl.BlockSpec((hdim, wd2), lambda i: (0, 0)),
        ],
        out_specs=pl.BlockSpec((blk, wd2), lambda i: (i, 0)),
        out_shape=jax.ShapeDtypeStruct((n, wd2), jnp.float32),
    )(p, dinv, b1[None, :], w2p)


def _tc3(q, dinv, b2p, blk):
    _, n, wd2 = q.shape
    grid = n // blk
    return pl.pallas_call(
        _tc3_body,
        grid=(grid,),
        in_specs=[
            pl.BlockSpec((NC, blk, wd2), lambda i: (0, i, 0)),
            pl.BlockSpec((blk, 1), lambda i: (i, 0)),
            pl.BlockSpec((1, wd2), lambda i: (0, 0)),
        ],
        out_specs=pl.BlockSpec((blk, wd2), lambda i: (i, 0)),
        out_shape=jax.ShapeDtypeStruct((n, wd2), jnp.float32),
    )(q, dinv, b2p[None, :])


def kernel(edge_index, edge_weight, emb, W1, b1, W2, b2):
    n, hdim = emb.shape
    ncls = W2.shape[1]
    e = edge_index.shape[1]

    loop = jnp.arange(n, dtype=jnp.int32)
    src = jnp.concatenate([edge_index[0].astype(jnp.int32), loop])
    dst = jnp.concatenate([edge_index[1].astype(jnp.int32), loop])
    ew = jnp.concatenate([edge_weight.astype(jnp.float32),
                          jnp.ones((n,), jnp.float32)])
    etot = e + n
    unit = NW * CH * RB
    epad = -(-etot // unit) * unit
    pad = epad - etot
    if pad:
        pidx = jnp.arange(pad, dtype=jnp.int32) % n
        src = jnp.concatenate([src, pidx])
        dst = jnp.concatenate([dst, pidx])
        ew = jnp.concatenate([ew, jnp.zeros((pad,), jnp.float32)])
    kmc = epad // (NW * CH)
    src3 = src.reshape(NW, kmc, CH)
    dst3 = dst.reshape(NW, kmc, CH)
    ew3 = ew.reshape(NW, kmc, CH)

    wd2 = 32
    w2p = jnp.zeros((hdim, wd2), jnp.float32).at[:, :ncls].set(W2)
    b2p = jnp.zeros((wd2,), jnp.float32).at[:ncls].set(b2)

    blk = 1000
    kdeg = epad // (NW * 128)
    degp = _make_deg(n, kdeg)(dst.reshape(NW, kdeg, 128),
                              ew.reshape(NW, kdeg, 128)).reshape(NC, n)
    h1s, dinv = _tc1(degp, emb, W1, blk)
    p = _make_agg(n, kmc, hdim // 2, 2)(src3, dst3, ew3, h1s)
    h2 = _tc2(p, dinv, b1, w2p, blk)
    q = _make_agg(n, kmc, wd2, 1)(src3, dst3, ew3, h2[None])
    out32 = _tc3(q[0], dinv, b2p, blk)
    return out32[:, :ncls]


# speedup vs baseline: 1.0205x; 1.0205x over previous
import functools

import jax
import jax.numpy as jnp
from jax import lax
from jax.experimental import pallas as pl
from jax.experimental.pallas import tpu as pltpu
from jax.experimental.pallas import tpu_sc as plsc

NC = 2
NS = 16
NW = NC * NS
CH = 256
RB = 3


def _mesh():
    return plsc.VectorSubcoreMesh(core_axis_name="c", subcore_axis_name="s")


def _make_deg(n, kmc):
    chunk = 1000
    assert chunk % 8 == 0 and n % chunk == 0
    ntz = n // chunk
    zlen = -(-chunk // 16) * 16

    def body(dst3, ew3, out, dst2, ew2, zbuf, acc):
        c = lax.axis_index("c")
        s = lax.axis_index("s")
        w = c * NS + s
        pltpu.sync_copy(dst3.at[w], dst2)
        pltpu.sync_copy(ew3.at[w], ew2)

        def zb(i, _):
            zbuf[pl.ds(i * 16, 16)] = jnp.zeros((16,), jnp.float32)
            return 0
        lax.fori_loop(0, zlen // 16, zb, 0, unroll=8)

        @pl.when(s < ntz)
        def _():
            pltpu.sync_copy(zbuf.at[pl.ds(0, chunk)],
                            acc.at[pl.ds(s * chunk, chunk)])
        plsc.subcore_barrier()

        def scat(j, _):
            pltpu.sync_copy(ew2.at[j], acc.at[dst2.at[j]], add=True)
            return 0
        lax.fori_loop(0, kmc, scat, 0)
        plsc.subcore_barrier()

        @pl.when(s < ntz)
        def _():
            pltpu.sync_copy(acc.at[pl.ds(s * chunk, chunk)],
                            zbuf.at[pl.ds(0, chunk)])
            pltpu.sync_copy(zbuf.at[pl.ds(0, chunk)],
                            out.at[pl.ds(c * n + s * chunk, chunk)])

    return pl.kernel(
        body,
        out_type=jax.ShapeDtypeStruct((NC * n,), jnp.float32),
        mesh=_mesh(),
        scratch_types=[
            pltpu.VMEM((kmc, 128), jnp.int32),
            pltpu.VMEM((kmc, 128), jnp.float32),
            pltpu.VMEM((zlen,), jnp.float32),
            pltpu.VMEM_SHARED((n,), jnp.float32),
        ],
    )


def _make_agg(n, kmc, wd, nh, stage_table=False):
    rows = (n // NS) // 8 * 8
    rem = n - NS * rows
    assert kmc % RB == 0 and rem % 8 == 0

    def body(src3, dst3, ew3, h, out, src2, dst2, ew2, b0, b1, b2, acc,
             *rest):
        if stage_table:
            tbl, gsem, ssem = rest
        else:
            gsem, ssem = rest
        c = lax.axis_index("c")
        s = lax.axis_index("s")
        w = c * NS + s
        bufs = (b0, b1, b2)
        pltpu.sync_copy(src3.at[w], src2)
        pltpu.sync_copy(dst3.at[w], dst2)
        pltpu.sync_copy(ew3.at[w], ew2)

        def zb(e, _):
            for g in range(wd // 16):
                b0[e, pl.ds(g * 16, 16)] = jnp.zeros((16,), jnp.float32)
            return 0
        lax.fori_loop(0, CH, zb, 0)

        def zero_acc():
            def zrange(base, nr):
                off = 0
                while off < nr:
                    sz = min(CH, nr - off)
                    pltpu.sync_copy(b0.at[pl.ds(0, sz)],
                                    acc.at[pl.ds(base + off, sz)])
                    off += sz
            zrange(s * rows, rows)

            @pl.when(s == 0)
            def _():
                zrange(NS * rows, rem)

        def scale(j, buf):
            ewrow = ew2.at[j]

            def tscale(t, _):
                ewv = ewrow[pl.ds(t * 16, 16)]
                bt = buf.at[pl.ds(t * 16, 16)]
                for i in range(16):
                    bv = ewv.at[jnp.full((16,), i, jnp.int32)].get(
                        mode="promise_in_bounds")
                    for g in range(wd // 16):
                        sl = (i, pl.ds(g * 16, 16))
                        bt[sl] = bt[sl] * bv
                return 0
            lax.fori_loop(0, CH // 16, tscale, 0, unroll=2)

        for f in range(nh):
            if stage_table:
                off = 0
                while off < rows:
                    sz = min(CH, rows - off)
                    pltpu.sync_copy(h.at[f, pl.ds(s * rows + off, sz)],
                                    b1.at[pl.ds(0, sz)])
                    pltpu.sync_copy(b1.at[pl.ds(0, sz)],
                                    tbl.at[pl.ds(s * rows + off, sz)])
                    off += sz

                @pl.when(s == 0)
                def _():
                    off2 = 0
                    while off2 < rem:
                        sz = min(CH, rem - off2)
                        pltpu.sync_copy(h.at[f, pl.ds(NS * rows + off2, sz)],
                                        b1.at[pl.ds(0, sz)])
                        pltpu.sync_copy(b1.at[pl.ds(0, sz)],
                                        tbl.at[pl.ds(NS * rows + off2, sz)])
                        off2 += sz
                hf = tbl
            else:
                hf = h.at[f]
            zero_acc()
            plsc.subcore_barrier()

            def gfire(j, buf):
                pltpu.async_copy(hf.at[src2.at[j]], buf, gsem)

            def gwait(j, buf):
                pltpu.make_async_copy(hf.at[src2.at[j]], buf, gsem).wait()

            def sfire(j, buf):
                pltpu.async_copy(buf, acc.at[dst2.at[j]], ssem, add=True)

            def swait(j, buf):
                pltpu.make_async_copy(buf, acc.at[dst2.at[j]], ssem).wait()

            gfire(0, b0)
            gfire(1, b1)

            def step(i, _):
                for k in range(RB):
                    j = RB * i + k
                    buf = bufs[k]
                    nbuf = bufs[(k + 2) % RB]
                    gwait(j, buf)
                    scale(j, buf)
                    sfire(j, buf)

                    @pl.when(jnp.logical_and(j >= 1, j + 2 < kmc))
                    def _():
                        swait(j - 1, nbuf)

                    @pl.when(j + 2 < kmc)
                    def _():
                        gfire(j + 2, nbuf)
                return 0
            lax.fori_loop(0, kmc // RB, step, 0)
            for j in range(kmc - RB, kmc):
                swait(j, bufs[j % RB])
            plsc.subcore_barrier()

            def dump(base, nr):
                off = 0
                while off < nr:
                    sz = min(CH, nr - off)
                    pltpu.sync_copy(acc.at[pl.ds(base + off, sz)],
                                    b0.at[pl.ds(0, sz)])
                    pltpu.sync_copy(b0.at[pl.ds(0, sz)],
                                    out.at[f, c, pl.ds(base + off, sz)])
                    off += sz
            dump(s * rows, rows)

            @pl.when(s == 0)
            def _():
                dump(NS * rows, rem)
            if f + 1 < nh:
                lax.fori_loop(0, CH, zb, 0)
                plsc.subcore_barrier()

    return pl.kernel(
        body,
        out_type=jax.ShapeDtypeStruct((nh, NC, n, wd), jnp.float32),
        mesh=_mesh(),
        compiler_params=pltpu.CompilerParams(use_tc_tiling_on_sc=False),
        scratch_types=[
            pltpu.VMEM((kmc, CH), jnp.int32),
            pltpu.VMEM((kmc, CH), jnp.int32),
            pltpu.VMEM((kmc, CH), jnp.float32),
            pltpu.VMEM((CH, wd), jnp.float32),
            pltpu.VMEM((CH, wd), jnp.float32),
            pltpu.VMEM((CH, wd), jnp.float32),
            pltpu.VMEM_SHARED((n, wd), jnp.float32),
        ] + ([pltpu.VMEM_SHARED((n, wd), jnp.float32)] if stage_table else [])
          + [
            pltpu.SemaphoreType.DMA,
            pltpu.SemaphoreType.DMA,
        ],
    )


_PREC = lax.Precision.HIGHEST


def _tc1_body(degp, emb, w1, h1s, dinv_out):
    deg = degp[0] + degp[1]
    dinv = jnp.where(deg > 0, lax.rsqrt(deg), 0.0)
    hm = jnp.dot(emb[...], w1[...], preferred_element_type=jnp.float32,
                 precision=_PREC)
    hm = hm * dinv
    half = hm.shape[1] // 2
    h1s[0] = hm[:, :half]
    h1s[1] = hm[:, half:]
    dinv_out[...] = dinv


def _tc2_body(p, dinv_in, b1, w2, h2):
    dinv = dinv_in[...]
    p01 = jnp.concatenate([p[0, 0] + p[0, 1], p[1, 0] + p[1, 1]], axis=1)
    x = p01 * dinv + b1[...]
    x = jnp.maximum(x, 0.0)
    hm = jnp.dot(x, w2[...], preferred_element_type=jnp.float32,
                 precision=_PREC)
    h2[...] = hm * dinv


def _tc3_body(q, dinv_in, b2, out):
    out[...] = (q[0] + q[1]) * dinv_in[...] + b2[...]


def _tc1(degp, emb, w1, blk):
    n, hdim = emb.shape
    grid = n // blk
    return pl.pallas_call(
        _tc1_body,
        grid=(grid,),
        in_specs=[
            pl.BlockSpec((NC, blk, 1), lambda i: (0, i, 0)),
            pl.BlockSpec((blk, hdim), lambda i: (i, 0)),
            pl.BlockSpec((hdim, hdim), lambda i: (0, 0)),
        ],
        out_specs=[
            pl.BlockSpec((2, blk, hdim // 2), lambda i: (0, i, 0)),
            pl.BlockSpec((blk, 1), lambda i: (i, 0)),
        ],
        out_shape=[
            jax.ShapeDtypeStruct((2, n, hdim // 2), jnp.float32),
            jax.ShapeDtypeStruct((n, 1), jnp.float32),
        ],
    )(degp[..., None], emb, w1)


def _tc2(p, dinv, b1, w2p, blk):
    _, _, n, hh = p.shape
    hdim = 2 * hh
    wd2 = w2p.shape[1]
    grid = n // blk
    return pl.pallas_call(
        _tc2_body,
        grid=(grid,),
        in_specs=[
            pl.BlockSpec((2, NC, blk, hh), lambda i: (0, 0, i, 0)),
            pl.BlockSpec((blk, 1), lambda i: (i, 0)),
            pl.BlockSpec((1, hdim), lambda i: (0, 0)),
            pl.BlockSpec((hdim, wd2), lambda i: (0, 0)),
        ],
        out_specs=pl.BlockSpec((blk, wd2), lambda i: (i, 0)),
        out_shape=jax.ShapeDtypeStruct((n, wd2), jnp.float32),
    )(p, dinv, b1[None, :], w2p)


def _tc3(q, dinv, b2p, blk):
    _, n, wd2 = q.shape
    grid = n // blk
    return pl.pallas_call(
        _tc3_body,
        grid=(grid,),
        in_specs=[
            pl.BlockSpec((NC, blk, wd2), lambda i: (0, i, 0)),
            pl.BlockSpec((blk, 1), lambda i: (i, 0)),
            pl.BlockSpec((1, wd2), lambda i: (0, 0)),
        ],
        out_specs=pl.BlockSpec((blk, wd2), lambda i: (i, 0)),
        out_shape=jax.ShapeDtypeStruct((n, wd2), jnp.float32),
    )(q, dinv, b2p[None, :])


def kernel(edge_index, edge_weight, emb, W1, b1, W2, b2):
    n, hdim = emb.shape
    ncls = W2.shape[1]
    e = edge_index.shape[1]

    loop = jnp.arange(n, dtype=jnp.int32)
    src = jnp.concatenate([edge_index[0].astype(jnp.int32), loop])
    dst = jnp.concatenate([edge_index[1].astype(jnp.int32), loop])
    ew = jnp.concatenate([edge_weight.astype(jnp.float32),
                          jnp.ones((n,), jnp.float32)])
    etot = e + n
    unit = NW * CH * RB
    epad = -(-etot // unit) * unit
    pad = epad - etot
    if pad:
        pidx = jnp.arange(pad, dtype=jnp.int32) % n
        src = jnp.concatenate([src, pidx])
        dst = jnp.concatenate([dst, pidx])
        ew = jnp.concatenate([ew, jnp.zeros((pad,), jnp.float32)])
    kmc = epad // (NW * CH)
    src3 = src.reshape(NW, kmc, CH)
    dst3 = dst.reshape(NW, kmc, CH)
    ew3 = ew.reshape(NW, kmc, CH)

    wd2 = 32
    w2p = jnp.zeros((hdim, wd2), jnp.float32).at[:, :ncls].set(W2)
    b2p = jnp.zeros((wd2,), jnp.float32).at[:ncls].set(b2)

    blk = 1000
    kdeg = epad // (NW * 128)
    degp = _make_deg(n, kdeg)(dst.reshape(NW, kdeg, 128),
                              ew.reshape(NW, kdeg, 128)).reshape(NC, n)
    h1s, dinv = _tc1(degp, emb, W1, blk)
    p = _make_agg(n, kmc, hdim // 2, 2)(src3, dst3, ew3, h1s)
    h2 = _tc2(p, dinv, b1, w2p, blk)
    q = _make_agg(n, kmc, wd2, 1, stage_table=True)(src3, dst3, ew3, h2[None])
    out32 = _tc3(q[0], dinv, b2p, blk)
    return out32[:, :ncls]


# speedup vs baseline: 1.0427x; 1.0218x over previous
import functools

import jax
import jax.numpy as jnp
from jax import lax
from jax.experimental import pallas as pl
from jax.experimental.pallas import tpu as pltpu
from jax.experimental.pallas import tpu_sc as plsc

NC = 2
NS = 16
NW = NC * NS
CH = 256
RB = 3


def _mesh():
    return plsc.VectorSubcoreMesh(core_axis_name="c", subcore_axis_name="s")


def _make_deg(n, kmc):
    chunk = 1000
    assert chunk % 8 == 0 and n % chunk == 0
    ntz = n // chunk
    zlen = -(-chunk // 16) * 16

    def body(dst3, ew3, out, dst2, ew2, zbuf, acc):
        c = lax.axis_index("c")
        s = lax.axis_index("s")
        w = c * NS + s
        pltpu.sync_copy(dst3.at[w], dst2)
        pltpu.sync_copy(ew3.at[w], ew2)

        def zb(i, _):
            zbuf[pl.ds(i * 16, 16)] = jnp.zeros((16,), jnp.float32)
            return 0
        lax.fori_loop(0, zlen // 16, zb, 0, unroll=8)

        @pl.when(s < ntz)
        def _():
            pltpu.sync_copy(zbuf.at[pl.ds(0, chunk)],
                            acc.at[pl.ds(s * chunk, chunk)])
        plsc.subcore_barrier()

        def scat(j, _):
            pltpu.sync_copy(ew2.at[j], acc.at[dst2.at[j]], add=True)
            return 0
        lax.fori_loop(0, kmc, scat, 0)
        plsc.subcore_barrier()

        @pl.when(s < ntz)
        def _():
            pltpu.sync_copy(acc.at[pl.ds(s * chunk, chunk)],
                            zbuf.at[pl.ds(0, chunk)])
            pltpu.sync_copy(zbuf.at[pl.ds(0, chunk)],
                            out.at[pl.ds(c * n + s * chunk, chunk)])

    return pl.kernel(
        body,
        out_type=jax.ShapeDtypeStruct((NC * n,), jnp.float32),
        mesh=_mesh(),
        scratch_types=[
            pltpu.VMEM((kmc, 128), jnp.int32),
            pltpu.VMEM((kmc, 128), jnp.float32),
            pltpu.VMEM((zlen,), jnp.float32),
            pltpu.VMEM_SHARED((n,), jnp.float32),
        ],
    )


def _make_agg(n, kmc, wd, nh, stage_table=False):
    rows = (n // NS) // 8 * 8
    rem = n - NS * rows
    assert kmc % RB == 0 and rem % 8 == 0

    def body(src3, dst3, ew3, h, out, src2, dst2, ew2, b0, b1, b2, acc,
             *rest):
        if stage_table:
            tbl, gsem, ssem = rest
        else:
            gsem, ssem = rest
        c = lax.axis_index("c")
        s = lax.axis_index("s")
        w = c * NS + s
        bufs = (b0, b1, b2)
        pltpu.sync_copy(src3.at[w], src2)
        pltpu.sync_copy(dst3.at[w], dst2)
        pltpu.sync_copy(ew3.at[w], ew2)

        def zb(e, _):
            for g in range(wd // 16):
                b0[e, pl.ds(g * 16, 16)] = jnp.zeros((16,), jnp.float32)
            return 0
        lax.fori_loop(0, CH, zb, 0)

        def zero_acc():
            def zrange(base, nr):
                off = 0
                while off < nr:
                    sz = min(CH, nr - off)
                    pltpu.sync_copy(b0.at[pl.ds(0, sz)],
                                    acc.at[pl.ds(base + off, sz)])
                    off += sz
            zrange(s * rows, rows)

            @pl.when(s == 0)
            def _():
                zrange(NS * rows, rem)

        def scale(j, buf):
            ewrow = ew2.at[j]

            def tscale(t, _):
                ewv = ewrow[pl.ds(t * 16, 16)]
                bt = buf.at[pl.ds(t * 16, 16)]
                for i in range(16):
                    bv = ewv.at[jnp.full((16,), i, jnp.int32)].get(
                        mode="promise_in_bounds")
                    for g in range(wd // 16):
                        sl = (i, pl.ds(g * 16, 16))
                        bt[sl] = bt[sl] * bv
                return 0
            lax.fori_loop(0, CH // 16, tscale, 0, unroll=2)

        for f in range(nh):
            if stage_table:
                off = 0
                while off < rows:
                    sz = min(CH, rows - off)
                    pltpu.sync_copy(h.at[f, pl.ds(s * rows + off, sz)],
                                    b1.at[pl.ds(0, sz)])
                    pltpu.sync_copy(b1.at[pl.ds(0, sz)],
                                    tbl.at[pl.ds(s * rows + off, sz)])
                    off += sz

                @pl.when(s == 0)
                def _():
                    off2 = 0
                    while off2 < rem:
                        sz = min(CH, rem - off2)
                        pltpu.sync_copy(h.at[f, pl.ds(NS * rows + off2, sz)],
                                        b1.at[pl.ds(0, sz)])
                        pltpu.sync_copy(b1.at[pl.ds(0, sz)],
                                        tbl.at[pl.ds(NS * rows + off2, sz)])
                        off2 += sz
                hf = tbl
            else:
                hf = h.at[f]
            zero_acc()
            plsc.subcore_barrier()

            def gfire(j, buf):
                pltpu.async_copy(hf.at[src2.at[j]], buf, gsem)

            def gwait(j, buf):
                pltpu.make_async_copy(hf.at[src2.at[j]], buf, gsem).wait()

            def sfire(j, buf):
                pltpu.async_copy(buf, acc.at[dst2.at[j]], ssem, add=True)

            def swait(j, buf):
                pltpu.make_async_copy(buf, acc.at[dst2.at[j]], ssem).wait()

            gfire(0, b0)
            gfire(1, b1)

            def step(i, _):
                for k in range(RB):
                    j = RB * i + k
                    buf = bufs[k]
                    nbuf = bufs[(k + 2) % RB]
                    gwait(j, buf)
                    scale(j, buf)
                    sfire(j, buf)

                    @pl.when(jnp.logical_and(j >= 1, j + 2 < kmc))
                    def _():
                        swait(j - 1, nbuf)

                    @pl.when(j + 2 < kmc)
                    def _():
                        gfire(j + 2, nbuf)
                return 0
            lax.fori_loop(0, kmc // RB, step, 0)
            for j in range(kmc - RB, kmc):
                swait(j, bufs[j % RB])
            plsc.subcore_barrier()

            def dump(base, nr):
                off = 0
                while off < nr:
                    sz = min(CH, nr - off)
                    pltpu.sync_copy(acc.at[pl.ds(base + off, sz)],
                                    b0.at[pl.ds(0, sz)])
                    pltpu.sync_copy(b0.at[pl.ds(0, sz)],
                                    out.at[f, c, pl.ds(base + off, sz)])
                    off += sz
            dump(s * rows, rows)

            @pl.when(s == 0)
            def _():
                dump(NS * rows, rem)
            if f + 1 < nh:
                lax.fori_loop(0, CH, zb, 0)
                plsc.subcore_barrier()

    return pl.kernel(
        body,
        out_type=jax.ShapeDtypeStruct((nh, NC, n, wd), jnp.float32),
        mesh=_mesh(),
        compiler_params=pltpu.CompilerParams(use_tc_tiling_on_sc=False),
        scratch_types=[
            pltpu.VMEM((kmc, CH), jnp.int32),
            pltpu.VMEM((kmc, CH), jnp.int32),
            pltpu.VMEM((kmc, CH), jnp.float32),
            pltpu.VMEM((CH, wd), jnp.float32),
            pltpu.VMEM((CH, wd), jnp.float32),
            pltpu.VMEM((CH, wd), jnp.float32),
            pltpu.VMEM_SHARED((n, wd), jnp.float32),
        ] + ([pltpu.VMEM_SHARED((n, wd), jnp.float32)] if stage_table else [])
          + [
            pltpu.SemaphoreType.DMA,
            pltpu.SemaphoreType.DMA,
        ],
    )


_PREC = lax.Precision.HIGHEST


def _tc1_body(degp, emb, w1, h1s, dinv_out):
    deg = degp[0] + degp[1] + 1.0
    dinv = jnp.where(deg > 0, lax.rsqrt(deg), 0.0)
    hm = jnp.dot(emb[...], w1[...], preferred_element_type=jnp.float32,
                 precision=_PREC)
    hm = hm * dinv
    half = hm.shape[1] // 2
    h1s[0] = hm[:, :half]
    h1s[1] = hm[:, half:]
    dinv_out[...] = dinv


def _tc2_body(p, h1s, dinv_in, b1, w2, h2):
    dinv = dinv_in[...]
    p01 = jnp.concatenate([p[0, 0] + p[0, 1] + h1s[0],
                           p[1, 0] + p[1, 1] + h1s[1]], axis=1)
    x = p01 * dinv + b1[...]
    x = jnp.maximum(x, 0.0)
    hm = jnp.dot(x, w2[...], preferred_element_type=jnp.float32,
                 precision=_PREC)
    h2[...] = hm * dinv


def _tc3_body(q, h2, dinv_in, b2, out):
    out[...] = (q[0] + q[1] + h2[...]) * dinv_in[...] + b2[...]


def _tc1(degp, emb, w1, blk):
    n, hdim = emb.shape
    grid = n // blk
    return pl.pallas_call(
        _tc1_body,
        grid=(grid,),
        in_specs=[
            pl.BlockSpec((NC, blk, 1), lambda i: (0, i, 0)),
            pl.BlockSpec((blk, hdim), lambda i: (i, 0)),
            pl.BlockSpec((hdim, hdim), lambda i: (0, 0)),
        ],
        out_specs=[
            pl.BlockSpec((2, blk, hdim // 2), lambda i: (0, i, 0)),
            pl.BlockSpec((blk, 1), lambda i: (i, 0)),
        ],
        out_shape=[
            jax.ShapeDtypeStruct((2, n, hdim // 2), jnp.float32),
            jax.ShapeDtypeStruct((n, 1), jnp.float32),
        ],
    )(degp[..., None], emb, w1)


def _tc2(p, h1s, dinv, b1, w2p, blk):
    _, _, n, hh = p.shape
    hdim = 2 * hh
    wd2 = w2p.shape[1]
    grid = n // blk
    return pl.pallas_call(
        _tc2_body,
        grid=(grid,),
        in_specs=[
            pl.BlockSpec((2, NC, blk, hh), lambda i: (0, 0, i, 0)),
            pl.BlockSpec((2, blk, hh), lambda i: (0, i, 0)),
            pl.BlockSpec((blk, 1), lambda i: (i, 0)),
            pl.BlockSpec((1, hdim), lambda i: (0, 0)),
            pl.BlockSpec((hdim, wd2), lambda i: (0, 0)),
        ],
        out_specs=pl.BlockSpec((blk, wd2), lambda i: (i, 0)),
        out_shape=jax.ShapeDtypeStruct((n, wd2), jnp.float32),
    )(p, h1s, dinv, b1[None, :], w2p)


def _tc3(q, h2, dinv, b2p, blk):
    _, n, wd2 = q.shape
    grid = n // blk
    return pl.pallas_call(
        _tc3_body,
        grid=(grid,),
        in_specs=[
            pl.BlockSpec((NC, blk, wd2), lambda i: (0, i, 0)),
            pl.BlockSpec((blk, wd2), lambda i: (i, 0)),
            pl.BlockSpec((blk, 1), lambda i: (i, 0)),
            pl.BlockSpec((1, wd2), lambda i: (0, 0)),
        ],
        out_specs=pl.BlockSpec((blk, wd2), lambda i: (i, 0)),
        out_shape=jax.ShapeDtypeStruct((n, wd2), jnp.float32),
    )(q, h2, dinv, b2p[None, :])


def kernel(edge_index, edge_weight, emb, W1, b1, W2, b2):
    n, hdim = emb.shape
    ncls = W2.shape[1]
    e = edge_index.shape[1]

    src = edge_index[0].astype(jnp.int32)
    dst = edge_index[1].astype(jnp.int32)
    ew = edge_weight.astype(jnp.float32)
    unit = NW * CH * RB
    epad = -(-e // unit) * unit
    pad = epad - e
    if pad:
        pidx = jnp.arange(pad, dtype=jnp.int32) % n
        src = jnp.concatenate([src, pidx])
        dst = jnp.concatenate([dst, pidx])
        ew = jnp.concatenate([ew, jnp.zeros((pad,), jnp.float32)])
    kmc = epad // (NW * CH)
    src3 = src.reshape(NW, kmc, CH)
    dst3 = dst.reshape(NW, kmc, CH)
    ew3 = ew.reshape(NW, kmc, CH)

    wd2 = 32
    w2p = jnp.zeros((hdim, wd2), jnp.float32).at[:, :ncls].set(W2)
    b2p = jnp.zeros((wd2,), jnp.float32).at[:ncls].set(b2)

    blk = 1000
    kdeg = epad // (NW * 128)
    degp = _make_deg(n, kdeg)(dst.reshape(NW, kdeg, 128),
                              ew.reshape(NW, kdeg, 128)).reshape(NC, n)
    h1s, dinv = _tc1(degp, emb, W1, blk)
    p = _make_agg(n, kmc, hdim // 2, 2)(src3, dst3, ew3, h1s)
    h2 = _tc2(p, h1s, dinv, b1, w2p, blk)
    q = _make_agg(n, kmc, wd2, 1)(src3, dst3, ew3, h2[None])
    out32 = _tc3(q[0], h2, dinv, b2p, blk)
    return out32[:, :ncls]


# speedup vs baseline: 1.2222x; 1.1722x over previous
import functools

import jax
import jax.numpy as jnp
from jax import lax
from jax.experimental import pallas as pl
from jax.experimental.pallas import tpu as pltpu
from jax.experimental.pallas import tpu_sc as plsc

NC = 2
NS = 16
NW = NC * NS
CH = 256
RB = 3


def _mesh():
    return plsc.VectorSubcoreMesh(core_axis_name="c", subcore_axis_name="s")


def _make_deg(n, kmc):
    chunk = 1000
    assert chunk % 8 == 0 and n % chunk == 0
    ntz = n // chunk
    zlen = -(-chunk // 16) * 16

    def body(dst3, ew3, out, dst2, ew2, zbuf, acc):
        c = lax.axis_index("c")
        s = lax.axis_index("s")
        w = c * NS + s
        pltpu.sync_copy(dst3.at[w], dst2)
        pltpu.sync_copy(ew3.at[w], ew2)

        def zb(i, _):
            zbuf[pl.ds(i * 16, 16)] = jnp.zeros((16,), jnp.float32)
            return 0
        lax.fori_loop(0, zlen // 16, zb, 0, unroll=8)

        @pl.when(s < ntz)
        def _():
            pltpu.sync_copy(zbuf.at[pl.ds(0, chunk)],
                            acc.at[pl.ds(s * chunk, chunk)])
        plsc.subcore_barrier()

        def scat(j, _):
            pltpu.sync_copy(ew2.at[j], acc.at[dst2.at[j]], add=True)
            return 0
        lax.fori_loop(0, kmc, scat, 0)
        plsc.subcore_barrier()

        @pl.when(s < ntz)
        def _():
            pltpu.sync_copy(acc.at[pl.ds(s * chunk, chunk)],
                            zbuf.at[pl.ds(0, chunk)])
            pltpu.sync_copy(zbuf.at[pl.ds(0, chunk)],
                            out.at[pl.ds(c * n + s * chunk, chunk)])

    return pl.kernel(
        body,
        out_type=jax.ShapeDtypeStruct((NC * n,), jnp.float32),
        mesh=_mesh(),
        scratch_types=[
            pltpu.VMEM((kmc, 128), jnp.int32),
            pltpu.VMEM((kmc, 128), jnp.float32),
            pltpu.VMEM((zlen,), jnp.float32),
            pltpu.VMEM_SHARED((n,), jnp.float32),
        ],
    )


def _make_agg(n, kmc, wd, nh, stage_table=False):
    rows = (n // NS) // 8 * 8
    rem = n - NS * rows
    assert kmc % RB == 0 and rem % 8 == 0

    def body(src3, dst3, ew3, h, out, src2, dst2, ew2, b0, b1, b2, acc,
             *rest):
        if stage_table:
            tbl, gsem, ssem = rest
        else:
            gsem, ssem = rest
        c = lax.axis_index("c")
        s = lax.axis_index("s")
        w = c * NS + s
        bufs = (b0, b1, b2)
        pltpu.sync_copy(src3.at[w], src2)
        pltpu.sync_copy(dst3.at[w], dst2)
        pltpu.sync_copy(ew3.at[w], ew2)

        def zb(e, _):
            for g in range(wd // 16):
                b0[e, pl.ds(g * 16, 16)] = jnp.zeros((16,), jnp.float32)
            return 0
        lax.fori_loop(0, CH, zb, 0)

        def zero_acc():
            def zrange(base, nr):
                off = 0
                while off < nr:
                    sz = min(CH, nr - off)
                    pltpu.sync_copy(b0.at[pl.ds(0, sz)],
                                    acc.at[pl.ds(base + off, sz)])
                    off += sz
            zrange(s * rows, rows)

            @pl.when(s == 0)
            def _():
                zrange(NS * rows, rem)

        def scale(j, buf):
            ewrow = ew2.at[j]

            def tscale(t, _):
                ewv = ewrow[pl.ds(t * 16, 16)]
                bt = buf.at[pl.ds(t * 16, 16)]
                for i in range(16):
                    bv = ewv.at[jnp.full((16,), i, jnp.int32)].get(
                        mode="promise_in_bounds")
                    for g in range(wd // 16):
                        sl = (i, pl.ds(g * 16, 16))
                        bt[sl] = bt[sl] * bv
                return 0
            lax.fori_loop(0, CH // 16, tscale, 0, unroll=4)

        for f in range(nh):
            if stage_table:
                off = 0
                while off < rows:
                    sz = min(CH, rows - off)
                    pltpu.sync_copy(h.at[f, pl.ds(s * rows + off, sz)],
                                    b1.at[pl.ds(0, sz)])
                    pltpu.sync_copy(b1.at[pl.ds(0, sz)],
                                    tbl.at[pl.ds(s * rows + off, sz)])
                    off += sz

                @pl.when(s == 0)
                def _():
                    off2 = 0
                    while off2 < rem:
                        sz = min(CH, rem - off2)
                        pltpu.sync_copy(h.at[f, pl.ds(NS * rows + off2, sz)],
                                        b1.at[pl.ds(0, sz)])
                        pltpu.sync_copy(b1.at[pl.ds(0, sz)],
                                        tbl.at[pl.ds(NS * rows + off2, sz)])
                        off2 += sz
                hf = tbl
            else:
                hf = h.at[f]
            zero_acc()
            plsc.subcore_barrier()

            def gfire(j, buf):
                pltpu.async_copy(hf.at[src2.at[j]], buf, gsem)

            def gwait(j, buf):
                pltpu.make_async_copy(hf.at[src2.at[j]], buf, gsem).wait()

            def sfire(j, buf):
                pltpu.async_copy(buf, acc.at[dst2.at[j]], ssem, add=True)

            def swait(j, buf):
                pltpu.make_async_copy(buf, acc.at[dst2.at[j]], ssem).wait()

            gfire(0, b0)
            gfire(1, b1)

            def step(i, _):
                for k in range(RB):
                    j = RB * i + k
                    buf = bufs[k]
                    nbuf = bufs[(k + 2) % RB]
                    gwait(j, buf)
                    scale(j, buf)
                    sfire(j, buf)

                    @pl.when(jnp.logical_and(j >= 1, j + 2 < kmc))
                    def _():
                        swait(j - 1, nbuf)

                    @pl.when(j + 2 < kmc)
                    def _():
                        gfire(j + 2, nbuf)
                return 0
            lax.fori_loop(0, kmc // RB, step, 0)
            for j in range(kmc - RB, kmc):
                swait(j, bufs[j % RB])
            plsc.subcore_barrier()

            def dump(base, nr):
                off = 0
                while off < nr:
                    sz = min(CH, nr - off)
                    pltpu.sync_copy(acc.at[pl.ds(base + off, sz)],
                                    b0.at[pl.ds(0, sz)])
                    pltpu.sync_copy(b0.at[pl.ds(0, sz)],
                                    out.at[f, c, pl.ds(base + off, sz)])
                    off += sz
            dump(s * rows, rows)

            @pl.when(s == 0)
            def _():
                dump(NS * rows, rem)
            if f + 1 < nh:
                lax.fori_loop(0, CH, zb, 0)
                plsc.subcore_barrier()

    return pl.kernel(
        body,
        out_type=jax.ShapeDtypeStruct((nh, NC, n, wd), jnp.float32),
        mesh=_mesh(),
        compiler_params=pltpu.CompilerParams(use_tc_tiling_on_sc=False),
        scratch_types=[
            pltpu.VMEM((kmc, CH), jnp.int32),
            pltpu.VMEM((kmc, CH), jnp.int32),
            pltpu.VMEM((kmc, CH), jnp.float32),
            pltpu.VMEM((CH, wd), jnp.float32),
            pltpu.VMEM((CH, wd), jnp.float32),
            pltpu.VMEM((CH, wd), jnp.float32),
            pltpu.VMEM_SHARED((n, wd), jnp.float32),
        ] + ([pltpu.VMEM_SHARED((n, wd), jnp.float32)] if stage_table else [])
          + [
            pltpu.SemaphoreType.DMA,
            pltpu.SemaphoreType.DMA,
        ],
    )


_PREC = lax.Precision.HIGHEST


def _tc1_body(degp, emb, w1, h1s, dinv_out):
    deg = degp[0] + degp[1] + 1.0
    dinv = jnp.where(deg > 0, lax.rsqrt(deg), 0.0)
    hm = jnp.dot(emb[...], w1[...], preferred_element_type=jnp.float32,
                 precision=_PREC)
    hm = hm * dinv
    half = hm.shape[1] // 2
    h1s[0] = hm[:, :half]
    h1s[1] = hm[:, half:]
    dinv_out[...] = dinv


def _tc2_body(p, h1s, dinv_in, b1, w2, h2):
    dinv = dinv_in[...]
    p01 = jnp.concatenate([p[0, 0] + p[0, 1] + h1s[0],
                           p[1, 0] + p[1, 1] + h1s[1]], axis=1)
    x = p01 * dinv + b1[...]
    x = jnp.maximum(x, 0.0)
    hm = jnp.dot(x, w2[...], preferred_element_type=jnp.float32,
                 precision=_PREC)
    h2[...] = hm * dinv


def _tc3_body(q, h2, dinv_in, b2, out):
    out[...] = (q[0] + q[1] + h2[...]) * dinv_in[...] + b2[...]


def _tc1(degp, emb, w1, blk):
    n, hdim = emb.shape
    grid = n // blk
    return pl.pallas_call(
        _tc1_body,
        grid=(grid,),
        in_specs=[
            pl.BlockSpec((NC, blk, 1), lambda i: (0, i, 0)),
            pl.BlockSpec((blk, hdim), lambda i: (i, 0)),
            pl.BlockSpec((hdim, hdim), lambda i: (0, 0)),
        ],
        out_specs=[
            pl.BlockSpec((2, blk, hdim // 2), lambda i: (0, i, 0)),
            pl.BlockSpec((blk, 1), lambda i: (i, 0)),
        ],
        out_shape=[
            jax.ShapeDtypeStruct((2, n, hdim // 2), jnp.float32),
            jax.ShapeDtypeStruct((n, 1), jnp.float32),
        ],
    )(degp[..., None], emb, w1)


def _tc2(p, h1s, dinv, b1, w2p, blk):
    _, _, n, hh = p.shape
    hdim = 2 * hh
    wd2 = w2p.shape[1]
    grid = n // blk
    return pl.pallas_call(
        _tc2_body,
        grid=(grid,),
        in_specs=[
            pl.BlockSpec((2, NC, blk, hh), lambda i: (0, 0, i, 0)),
            pl.BlockSpec((2, blk, hh), lambda i: (0, i, 0)),
            pl.BlockSpec((blk, 1), lambda i: (i, 0)),
            pl.BlockSpec((1, hdim), lambda i: (0, 0)),
            pl.BlockSpec((hdim, wd2), lambda i: (0, 0)),
        ],
        out_specs=pl.BlockSpec((blk, wd2), lambda i: (i, 0)),
        out_shape=jax.ShapeDtypeStruct((n, wd2), jnp.float32),
    )(p, h1s, dinv, b1[None, :], w2p)


def _tc3(q, h2, dinv, b2p, blk):
    _, n, wd2 = q.shape
    grid = n // blk
    return pl.pallas_call(
        _tc3_body,
        grid=(grid,),
        in_specs=[
            pl.BlockSpec((NC, blk, wd2), lambda i: (0, i, 0)),
            pl.BlockSpec((blk, wd2), lambda i: (i, 0)),
            pl.BlockSpec((blk, 1), lambda i: (i, 0)),
            pl.BlockSpec((1, wd2), lambda i: (0, 0)),
        ],
        out_specs=pl.BlockSpec((blk, wd2), lambda i: (i, 0)),
        out_shape=jax.ShapeDtypeStruct((n, wd2), jnp.float32),
    )(q, h2, dinv, b2p[None, :])


def kernel(edge_index, edge_weight, emb, W1, b1, W2, b2):
    n, hdim = emb.shape
    ncls = W2.shape[1]
    e = edge_index.shape[1]

    src = edge_index[0].astype(jnp.int32)
    dst = edge_index[1].astype(jnp.int32)
    ew = edge_weight.astype(jnp.float32)
    unit = NW * CH * RB
    epad = -(-e // unit) * unit
    pad = epad - e
    if pad:
        pidx = jnp.arange(pad, dtype=jnp.int32) % n
        src = jnp.concatenate([src, pidx])
        dst = jnp.concatenate([dst, pidx])
        ew = jnp.concatenate([ew, jnp.zeros((pad,), jnp.float32)])
    kmc = epad // (NW * CH)
    src3 = src.reshape(NW, kmc, CH)
    dst3 = dst.reshape(NW, kmc, CH)
    ew3 = ew.reshape(NW, kmc, CH)

    wd2 = 32
    w2p = jnp.zeros((hdim, wd2), jnp.float32).at[:, :ncls].set(W2)
    b2p = jnp.zeros((wd2,), jnp.float32).at[:ncls].set(b2)

    blk = 1000
    kdeg = epad // (NW * 128)
    degp = _make_deg(n, kdeg)(dst.reshape(NW, kdeg, 128),
                              ew.reshape(NW, kdeg, 128)).reshape(NC, n)
    h1s, dinv = _tc1(degp, emb, W1, blk)
    p = _make_agg(n, kmc, hdim // 2, 2)(src3, dst3, ew3, h1s)
    h2 = _tc2(p, h1s, dinv, b1, w2p, blk)
    q = _make_agg(n, kmc, wd2, 1)(src3, dst3, ew3, h2[None])
    out32 = _tc3(q[0], h2, dinv, b2p, blk)
    return out32[:, :ncls]
